# baseline (speedup 1.0000x reference)
import functools

import jax
import jax.numpy as jnp
from jax import lax
from jax.experimental import pallas as pl
from jax.experimental.pallas import tpu as pltpu
from jax.experimental.pallas import tpu_sc as plsc

B, D, H, K = 32, 2048, 32768, 32
LAM = 0.001
BH = 512
NB = H // BH

_MATMUL_PREC = jax.lax.Precision.DEFAULT


_BLK = 128
_NBLK = H // _BLK


def _enc_kernel(x_ref, pb_ref, w_ref, be_ref, po_ref, enc_ref, pen_ref,
                bc_ref, pre_scr, key_scr):
    i = pl.program_id(0)
    xc = x_ref[...] - pb_ref[...]
    h = jax.lax.dot_general(xc, w_ref[...], (((1,), (0,)), ((), ())),
                            preferred_element_type=jnp.float32,
                            precision=_MATMUL_PREC)
    h = h + be_ref[...] + po_ref[...]
    pre_scr[:, pl.ds(i * BH, BH)] = h
    bits = jax.lax.bitcast_convert_type(h, jnp.int32)
    ks = jnp.where(bits >= 0, bits, bits ^ jnp.int32(0x7FFFFFFF))
    key_scr[:, pl.ds(i * BH, BH)] = ks

    @pl.when(i == NB - 1)
    def _():
        keys = key_scr[...]

        def body(j, t):
            shift = (31 - j).astype(jnp.int32)
            cand = t + (jnp.int32(1) << shift)
            cnt = jnp.sum((keys >= cand).astype(jnp.int32), axis=1,
                          keepdims=True)
            return jnp.where(cnt >= K, cand, t)

        t0 = jnp.full((B, 1), jnp.iinfo(jnp.int32).min, jnp.int32)
        t = jax.lax.fori_loop(0, 32, body, t0, unroll=True)
        pre = pre_scr[...]
        mask = (keys >= t) & (pre > 0.0)
        enc = jnp.where(mask, pre, 0.0)
        enc_ref[...] = enc
        pen_ref[...] = (jnp.sum(enc) * (LAM / (B * H))).reshape(1, 1)
        mf = jnp.where(mask, 1.0, 0.0)
        eind = (lax.broadcasted_iota(jnp.int32, (BH, BH // _BLK), 0) // _BLK
                == lax.broadcasted_iota(jnp.int32, (BH, BH // _BLK), 1)
                ).astype(jnp.float32)
        for j in range(NB):
            bc_ref[:, (BH // _BLK) * j:(BH // _BLK) * (j + 1)] = (
                jax.lax.dot_general(mf[:, BH * j:BH * (j + 1)], eind,
                                    (((1,), (0,)), ((), ())),
                                    preferred_element_type=jnp.float32))


def _dec_kernel(enc_ref, wd_ref, bd_ref, db_ref, out_ref, acc_scr):
    i = pl.program_id(0)

    @pl.when(i == 0)
    def _():
        acc_scr[...] = jnp.zeros_like(acc_scr)

    acc_scr[...] += jax.lax.dot_general(
        enc_ref[...], wd_ref[...], (((1,), (0,)), ((), ())),
        preferred_element_type=jnp.float32, precision=_MATMUL_PREC)

    @pl.when(i == NB - 1)
    def _():
        out_ref[...] = acc_scr[...] + bd_ref[...] + db_ref[...]


def _encode(x, W_enc, b_enc, pre_bias, post_bias, interpret=False):
    enc, pen, bc = pl.pallas_call(
        _enc_kernel,
        grid=(NB,),
        in_specs=[
            pl.BlockSpec((B, D), lambda i: (0, 0)),
            pl.BlockSpec((1, D), lambda i: (0, 0)),
            pl.BlockSpec((D, BH), lambda i: (0, i)),
            pl.BlockSpec((1, BH), lambda i: (0, i)),
            pl.BlockSpec((1, BH), lambda i: (0, i)),
        ],
        out_specs=[
            pl.BlockSpec((B, H), lambda i: (0, 0)),
            pl.BlockSpec((1, 1), lambda i: (0, 0)),
            pl.BlockSpec((B, _NBLK), lambda i: (0, 0)),
        ],
        out_shape=[
            jax.ShapeDtypeStruct((B, H), jnp.float32),
            jax.ShapeDtypeStruct((1, 1), jnp.float32),
            jax.ShapeDtypeStruct((B, _NBLK), jnp.float32),
        ],
        scratch_shapes=[
            pltpu.VMEM((B, H), jnp.float32),
            pltpu.VMEM((B, H), jnp.int32),
        ],
        compiler_params=pltpu.CompilerParams(
            dimension_semantics=("arbitrary",)),
        interpret=interpret,
    )(x, pre_bias.reshape(1, D), W_enc, b_enc.reshape(1, H),
      post_bias.reshape(1, H))
    return enc, pen, bc


def _decode_dense(enc, W_dec, b_dec, dec_bias, interpret=False):
    return pl.pallas_call(
        _dec_kernel,
        grid=(NB,),
        in_specs=[
            pl.BlockSpec((B, BH), lambda i: (0, i)),
            pl.BlockSpec((BH, D), lambda i: (i, 0)),
            pl.BlockSpec((1, D), lambda i: (0, 0)),
            pl.BlockSpec((1, D), lambda i: (0, 0)),
        ],
        out_specs=pl.BlockSpec((B, D), lambda i: (0, 0)),
        out_shape=jax.ShapeDtypeStruct((B, D), jnp.float32),
        scratch_shapes=[pltpu.VMEM((B, D), jnp.float32)],
        compiler_params=pltpu.CompilerParams(
            dimension_semantics=("arbitrary",)),
        interpret=interpret,
    )(enc, W_dec, b_dec.reshape(1, D), dec_bias.reshape(1, D))


_L = 16
_CAP = 48


def _sc_dec_body(enc2_hbm, wd_hbm, bd_hbm, db_hbm, bc_hbm, out_hbm,
                 seg_v, idx_v, val_v, blk_v, base_v, bc_v,
                 rows_a, rows_b, acc_v, bd_v, db_v, sem, sem_b):
    row = lax.axis_index("s") * 2 + lax.axis_index("c")
    pltpu.sync_copy(bc_hbm.at[row], bc_v)
    pltpu.sync_copy(bd_hbm, bd_v)
    pltpu.sync_copy(db_hbm, db_v)
    zi = jnp.zeros((_L,), jnp.int32)
    for j in range(_CAP // _L):
        idx_v[pl.ds(j * _L, _L)] = zi
        val_v[pl.ds(j * _L, _L)] = jnp.zeros((_L,), jnp.float32)
        blk_v[pl.ds(j * _L, _L)] = zi + row * _NBLK
        base_v[pl.ds(j * _L, _L)] = zi

    def l1_body(i, nb):
        c = bc_v[pl.ds(i * _L, _L)]
        m = c > 0.5
        loc = lax.iota(jnp.int32, _L) + i * _L
        plsc.store_compressed(blk_v.at[pl.ds(nb, _L)], loc + row * _NBLK,
                              mask=m)
        plsc.store_compressed(base_v.at[pl.ds(nb, _L)], loc * _BLK, mask=m)
        pc = plsc.all_reduce_population_count(m)[0]
        return jnp.minimum(nb + pc, _CAP - _L)

    nblk = lax.fori_loop(0, _NBLK // _L, l1_body, jnp.int32(0))

    pltpu.async_copy(enc2_hbm.at[blk_v.at[pl.ds(0, K)]], seg_v, sem).wait()

    def l2_body(s, cnt):
        ch = base_v[pl.ds((s // _L) * _L, _L)]
        lane = jnp.full((_L,), s % _L, jnp.int32)
        base = ch.at[lane].get(mode="promise_in_bounds")[0]

        def inner(q, cnt):
            v = seg_v[s, pl.ds(q * _L, _L)]
            m = v > 0.0
            plsc.store_compressed(val_v.at[pl.ds(cnt, _L)], v, mask=m)
            plsc.store_compressed(idx_v.at[pl.ds(cnt, _L)],
                                  lax.iota(jnp.int32, _L) + base + q * _L,
                                  mask=m)
            pc = plsc.all_reduce_population_count(m)[0]
            return jnp.minimum(cnt + pc, _CAP - _L)

        return lax.fori_loop(0, _BLK // _L, inner, cnt, unroll=True)

    lax.fori_loop(0, nblk, l2_body, jnp.int32(0))

    cp_a = pltpu.make_async_copy(wd_hbm.at[idx_v.at[pl.ds(0, K // 2)]],
                                 rows_a, sem)
    cp_b = pltpu.make_async_copy(wd_hbm.at[idx_v.at[pl.ds(K // 2, K // 2)]],
                                 rows_b, sem_b)
    cp_a.start()
    cp_b.start()

    vals = [val_v[pl.ds(j * _L, _L)][i]
            for j in range(K // _L) for i in range(_L)]

    cp_a.wait()

    def acc_a(c, carry):
        a = bd_v[pl.ds(c * _L, _L)] + db_v[pl.ds(c * _L, _L)]
        for k in range(K // 2):
            a = a + vals[k] * rows_a[k, pl.ds(c * _L, _L)]
        acc_v[pl.ds(c * _L, _L)] = a
        return carry

    lax.fori_loop(0, D // _L, acc_a, 0)
    cp_b.wait()

    def acc_b(c, carry):
        a = acc_v[pl.ds(c * _L, _L)]
        for k in range(K // 2):
            a = a + vals[K // 2 + k] * rows_b[k, pl.ds(c * _L, _L)]
        acc_v[pl.ds(c * _L, _L)] = a
        return carry

    lax.fori_loop(0, D // _L, acc_b, 0)
    pltpu.sync_copy(acc_v, out_hbm.at[row])


def _decode_sparse(enc, W_dec, b_dec, dec_bias, bc):
    mesh = plsc.VectorSubcoreMesh(core_axis_name="c", subcore_axis_name="s")
    f = functools.partial(
        pl.kernel,
        out_type=jax.ShapeDtypeStruct((B, D), jnp.float32),
        mesh=mesh,
        scratch_types=[
            pltpu.VMEM((K, _BLK), jnp.float32),
            pltpu.VMEM((_CAP,), jnp.int32),
            pltpu.VMEM((_CAP,), jnp.float32),
            pltpu.VMEM((_CAP,), jnp.int32),
            pltpu.VMEM((_CAP,), jnp.int32),
            pltpu.VMEM((_NBLK,), jnp.float32),
            pltpu.VMEM((K // 2, D), jnp.float32),
            pltpu.VMEM((K // 2, D), jnp.float32),
            pltpu.VMEM((D,), jnp.float32),
            pltpu.VMEM((D,), jnp.float32),
            pltpu.VMEM((D,), jnp.float32),
            pltpu.SemaphoreType.DMA,
            pltpu.SemaphoreType.DMA,
        ],
        compiler_params=pltpu.CompilerParams(needs_layout_passes=False),
    )(_sc_dec_body)
    return f(enc.reshape(B * _NBLK, _BLK), W_dec, b_dec, dec_bias, bc)


def kernel(x, W_enc, b_enc, pre_bias, post_bias, W_dec, b_dec, dec_bias):
    enc, pen, bc = _encode(x, W_enc, b_enc, pre_bias, post_bias)
    decoded = _decode_sparse(enc, W_dec, b_dec, dec_bias, bc)
    return decoded, enc, pen.reshape(())


# speedup vs baseline: 1.0275x; 1.0275x over previous
import functools

import jax
import jax.numpy as jnp
from jax import lax
from jax.experimental import pallas as pl
from jax.experimental.pallas import tpu as pltpu
from jax.experimental.pallas import tpu_sc as plsc

B, D, H, K = 32, 2048, 32768, 32
LAM = 0.001
BH = 512
NB = H // BH

_MATMUL_PREC = jax.lax.Precision.DEFAULT


_BLK = 128
_NBLK = H // _BLK


def _enc_kernel(x_ref, pb_ref, w_ref, be_ref, po_ref, enc_ref, pen_ref,
                bc_ref, pre_scr, key_scr):
    i = pl.program_id(0)
    xc = x_ref[...] - pb_ref[...]
    h = jax.lax.dot_general(xc, w_ref[...], (((1,), (0,)), ((), ())),
                            preferred_element_type=jnp.float32,
                            precision=_MATMUL_PREC)
    h = h + be_ref[...] + po_ref[...]
    pre_scr[:, pl.ds(i * BH, BH)] = h
    bits = jax.lax.bitcast_convert_type(h, jnp.int32)
    ks = jnp.where(bits >= 0, bits, bits ^ jnp.int32(0x7FFFFFFF))
    key_scr[:, pl.ds(i * BH, BH)] = ks

    @pl.when(i == NB - 1)
    def _():
        keys = key_scr[...]

        def body(j, t):
            shift = (31 - j).astype(jnp.int32)
            cand = t + (jnp.int32(1) << shift)
            cnt = jnp.sum((keys >= cand).astype(jnp.int32), axis=1,
                          keepdims=True)
            return jnp.where(cnt >= K, cand, t)

        t0 = jnp.full((B, 1), jnp.iinfo(jnp.int32).min, jnp.int32)
        t = jax.lax.fori_loop(0, 32, body, t0, unroll=True)
        pre = pre_scr[...]
        mask = (keys >= t) & (pre > 0.0)
        enc = jnp.where(mask, pre, 0.0)
        enc_ref[...] = enc
        pen_ref[...] = (jnp.sum(enc) * (LAM / (B * H))).reshape(1, 1)
        mf = jnp.where(mask, 1.0, 0.0)
        eind = (lax.broadcasted_iota(jnp.int32, (BH, BH // _BLK), 0) // _BLK
                == lax.broadcasted_iota(jnp.int32, (BH, BH // _BLK), 1)
                ).astype(jnp.float32)
        for j in range(NB):
            bc_ref[:, (BH // _BLK) * j:(BH // _BLK) * (j + 1)] = (
                jax.lax.dot_general(mf[:, BH * j:BH * (j + 1)], eind,
                                    (((1,), (0,)), ((), ())),
                                    preferred_element_type=jnp.float32))


def _dec_kernel(enc_ref, wd_ref, bd_ref, db_ref, out_ref, acc_scr):
    i = pl.program_id(0)

    @pl.when(i == 0)
    def _():
        acc_scr[...] = jnp.zeros_like(acc_scr)

    acc_scr[...] += jax.lax.dot_general(
        enc_ref[...], wd_ref[...], (((1,), (0,)), ((), ())),
        preferred_element_type=jnp.float32, precision=_MATMUL_PREC)

    @pl.when(i == NB - 1)
    def _():
        out_ref[...] = acc_scr[...] + bd_ref[...] + db_ref[...]


def _encode(x, W_enc, b_enc, pre_bias, post_bias, interpret=False):
    enc, pen, bc = pl.pallas_call(
        _enc_kernel,
        grid=(NB,),
        in_specs=[
            pl.BlockSpec((B, D), lambda i: (0, 0)),
            pl.BlockSpec((1, D), lambda i: (0, 0)),
            pl.BlockSpec((D, BH), lambda i: (0, i)),
            pl.BlockSpec((1, BH), lambda i: (0, i)),
            pl.BlockSpec((1, BH), lambda i: (0, i)),
        ],
        out_specs=[
            pl.BlockSpec((B, H), lambda i: (0, 0)),
            pl.BlockSpec((1, 1), lambda i: (0, 0)),
            pl.BlockSpec((B, _NBLK), lambda i: (0, 0)),
        ],
        out_shape=[
            jax.ShapeDtypeStruct((B, H), jnp.float32),
            jax.ShapeDtypeStruct((1, 1), jnp.float32),
            jax.ShapeDtypeStruct((B, _NBLK), jnp.float32),
        ],
        scratch_shapes=[
            pltpu.VMEM((B, H), jnp.float32),
            pltpu.VMEM((B, H), jnp.int32),
        ],
        compiler_params=pltpu.CompilerParams(
            dimension_semantics=("arbitrary",)),
        interpret=interpret,
    )(x, pre_bias.reshape(1, D), W_enc, b_enc.reshape(1, H),
      post_bias.reshape(1, H))
    return enc, pen, bc


def _decode_dense(enc, W_dec, b_dec, dec_bias, interpret=False):
    return pl.pallas_call(
        _dec_kernel,
        grid=(NB,),
        in_specs=[
            pl.BlockSpec((B, BH), lambda i: (0, i)),
            pl.BlockSpec((BH, D), lambda i: (i, 0)),
            pl.BlockSpec((1, D), lambda i: (0, 0)),
            pl.BlockSpec((1, D), lambda i: (0, 0)),
        ],
        out_specs=pl.BlockSpec((B, D), lambda i: (0, 0)),
        out_shape=jax.ShapeDtypeStruct((B, D), jnp.float32),
        scratch_shapes=[pltpu.VMEM((B, D), jnp.float32)],
        compiler_params=pltpu.CompilerParams(
            dimension_semantics=("arbitrary",)),
        interpret=interpret,
    )(enc, W_dec, b_dec.reshape(1, D), dec_bias.reshape(1, D))


_L = 16
_CAP = 48


def _sc_dec_body(enc_hbm, wd_hbm, bd_hbm, db_hbm, bc_hbm, out_hbm,
                 row_v, idx_v, val_v, blk_v, bc_v, rows_v, rows_b,
                 bd_v, db_v, sem, sem_b):
    row = lax.axis_index("s") * 2 + lax.axis_index("c")
    cp_row = pltpu.make_async_copy(enc_hbm.at[row], row_v, sem_b)
    cp_row.start()
    pltpu.sync_copy(bc_hbm.at[row], bc_v)
    pltpu.sync_copy(bd_hbm, bd_v)
    pltpu.sync_copy(db_hbm, db_v)
    zi = jnp.zeros((_L,), jnp.int32)
    for j in range(_CAP // _L):
        idx_v[pl.ds(j * _L, _L)] = zi
        val_v[pl.ds(j * _L, _L)] = jnp.zeros((_L,), jnp.float32)

    def l1_body(i, nb):
        c = bc_v[pl.ds(i * _L, _L)]
        m = c > 0.5
        plsc.store_compressed(blk_v.at[pl.ds(nb, _L)],
                              lax.iota(jnp.int32, _L) + i * _L, mask=m)
        pc = plsc.all_reduce_population_count(m)[0]
        return jnp.minimum(nb + pc, _CAP - _L)

    nblk = lax.fori_loop(0, _NBLK // _L, l1_body, jnp.int32(0))
    cp_row.wait()

    def l2_body(j, cnt):
        ch = blk_v[pl.ds((j // _L) * _L, _L)]
        sel = jnp.full((_L,), j % _L, jnp.int32)
        bid = ch.at[sel].get(mode="promise_in_bounds")[0]
        base = bid * _BLK

        def inner(q, cnt):
            v = row_v[pl.ds(base + q * _L, _L)]
            m = v > 0.0
            plsc.store_compressed(val_v.at[pl.ds(cnt, _L)], v, mask=m)
            plsc.store_compressed(idx_v.at[pl.ds(cnt, _L)],
                                  lax.iota(jnp.int32, _L) + base + q * _L,
                                  mask=m)
            pc = plsc.all_reduce_population_count(m)[0]
            return jnp.minimum(cnt + pc, _CAP - _L)

        return lax.fori_loop(0, _BLK // _L, inner, cnt, unroll=True)

    lax.fori_loop(0, nblk, l2_body, jnp.int32(0))

    cp_a = pltpu.make_async_copy(wd_hbm.at[idx_v.at[pl.ds(0, K // 2)]],
                                 rows_v, sem)
    cp_b = pltpu.make_async_copy(wd_hbm.at[idx_v.at[pl.ds(K // 2, K // 2)]],
                                 rows_b, sem_b)
    cp_a.start()
    cp_b.start()

    vals = [val_v[pl.ds(j * _L, _L)][i]
            for j in range(K // _L) for i in range(_L)]

    cp_a.wait()

    def acc_a(c, carry):
        a = bd_v[pl.ds(c * _L, _L)] + db_v[pl.ds(c * _L, _L)]
        for k in range(K // 2):
            a = a + vals[k] * rows_v[k, pl.ds(c * _L, _L)]
        row_v[pl.ds(c * _L, _L)] = a
        return carry

    lax.fori_loop(0, D // _L, acc_a, 0)
    cp_b.wait()

    def acc_b(c, carry):
        a = row_v[pl.ds(c * _L, _L)]
        for k in range(K // 2):
            a = a + vals[K // 2 + k] * rows_b[k, pl.ds(c * _L, _L)]
        row_v[pl.ds(c * _L, _L)] = a
        return carry

    lax.fori_loop(0, D // _L, acc_b, 0)
    pltpu.sync_copy(row_v.at[pl.ds(0, D)], out_hbm.at[row])


def _decode_sparse(enc, W_dec, b_dec, dec_bias, bc):
    mesh = plsc.VectorSubcoreMesh(core_axis_name="c", subcore_axis_name="s")
    f = functools.partial(
        pl.kernel,
        out_type=jax.ShapeDtypeStruct((B, D), jnp.float32),
        mesh=mesh,
        scratch_types=[
            pltpu.VMEM((H,), jnp.float32),
            pltpu.VMEM((_CAP,), jnp.int32),
            pltpu.VMEM((_CAP,), jnp.float32),
            pltpu.VMEM((_CAP,), jnp.int32),
            pltpu.VMEM((_NBLK,), jnp.float32),
            pltpu.VMEM((K // 2, D), jnp.float32),
            pltpu.VMEM((K // 2, D), jnp.float32),
            pltpu.VMEM((D,), jnp.float32),
            pltpu.VMEM((D,), jnp.float32),
            pltpu.SemaphoreType.DMA,
            pltpu.SemaphoreType.DMA,
        ],
        compiler_params=pltpu.CompilerParams(needs_layout_passes=False),
    )(_sc_dec_body)
    return f(enc, W_dec, b_dec, dec_bias, bc)


def kernel(x, W_enc, b_enc, pre_bias, post_bias, W_dec, b_dec, dec_bias):
    enc, pen, bc = _encode(x, W_enc, b_enc, pre_bias, post_bias)
    decoded = _decode_sparse(enc, W_dec, b_dec, dec_bias, bc)
    return decoded, enc, pen.reshape(())


# speedup vs baseline: 1.1024x; 1.0729x over previous
import functools

import jax
import jax.numpy as jnp
from jax import lax
from jax.experimental import pallas as pl
from jax.experimental.pallas import tpu as pltpu
from jax.experimental.pallas import tpu_sc as plsc

B, D, H, K = 32, 2048, 32768, 32
LAM = 0.001
BH = 1024
NB = H // BH

_MATMUL_PREC = jax.lax.Precision.DEFAULT


_BLK = 128
_NBLK = H // _BLK


def _enc_kernel(x_ref, pb_ref, w_ref, be_ref, po_ref, enc_ref, pen_ref,
                bc_ref, pre_scr, key_scr):
    i = pl.program_id(0)
    xc = x_ref[...] - pb_ref[...]
    h = jax.lax.dot_general(xc, w_ref[...], (((1,), (0,)), ((), ())),
                            preferred_element_type=jnp.float32,
                            precision=_MATMUL_PREC)
    h = h + be_ref[...] + po_ref[...]
    pre_scr[:, pl.ds(i * BH, BH)] = h
    bits = jax.lax.bitcast_convert_type(h, jnp.int32)
    ks = jnp.where(bits >= 0, bits, bits ^ jnp.int32(0x7FFFFFFF))
    key_scr[:, pl.ds(i * BH, BH)] = ks

    @pl.when(i == NB - 1)
    def _():
        keys = key_scr[...]

        def body(j, t):
            shift = (31 - j).astype(jnp.int32)
            cand = t + (jnp.int32(1) << shift)
            cnt = jnp.sum((keys >= cand).astype(jnp.int32), axis=1,
                          keepdims=True)
            return jnp.where(cnt >= K, cand, t)

        t0 = jnp.full((B, 1), jnp.iinfo(jnp.int32).min, jnp.int32)
        t = jax.lax.fori_loop(0, 32, body, t0, unroll=True)
        pre = pre_scr[...]
        mask = (keys >= t) & (pre > 0.0)
        enc = jnp.where(mask, pre, 0.0)
        enc_ref[...] = enc
        pen_ref[...] = (jnp.sum(enc) * (LAM / (B * H))).reshape(1, 1)
        mf = jnp.where(mask, 1.0, 0.0)
        eind = (lax.broadcasted_iota(jnp.int32, (BH, BH // _BLK), 0) // _BLK
                == lax.broadcasted_iota(jnp.int32, (BH, BH // _BLK), 1)
                ).astype(jnp.float32)
        for j in range(NB):
            bc_ref[:, (BH // _BLK) * j:(BH // _BLK) * (j + 1)] = (
                jax.lax.dot_general(mf[:, BH * j:BH * (j + 1)], eind,
                                    (((1,), (0,)), ((), ())),
                                    preferred_element_type=jnp.float32))


def _dec_kernel(enc_ref, wd_ref, bd_ref, db_ref, out_ref, acc_scr):
    i = pl.program_id(0)

    @pl.when(i == 0)
    def _():
        acc_scr[...] = jnp.zeros_like(acc_scr)

    acc_scr[...] += jax.lax.dot_general(
        enc_ref[...], wd_ref[...], (((1,), (0,)), ((), ())),
        preferred_element_type=jnp.float32, precision=_MATMUL_PREC)

    @pl.when(i == NB - 1)
    def _():
        out_ref[...] = acc_scr[...] + bd_ref[...] + db_ref[...]


def _encode(x, W_enc, b_enc, pre_bias, post_bias, interpret=False):
    enc, pen, bc = pl.pallas_call(
        _enc_kernel,
        grid=(NB,),
        in_specs=[
            pl.BlockSpec((B, D), lambda i: (0, 0)),
            pl.BlockSpec((1, D), lambda i: (0, 0)),
            pl.BlockSpec((D, BH), lambda i: (0, i)),
            pl.BlockSpec((1, BH), lambda i: (0, i)),
            pl.BlockSpec((1, BH), lambda i: (0, i)),
        ],
        out_specs=[
            pl.BlockSpec((B, H), lambda i: (0, 0)),
            pl.BlockSpec((1, 1), lambda i: (0, 0)),
            pl.BlockSpec((B, _NBLK), lambda i: (0, 0)),
        ],
        out_shape=[
            jax.ShapeDtypeStruct((B, H), jnp.float32),
            jax.ShapeDtypeStruct((1, 1), jnp.float32),
            jax.ShapeDtypeStruct((B, _NBLK), jnp.float32),
        ],
        scratch_shapes=[
            pltpu.VMEM((B, H), jnp.float32),
            pltpu.VMEM((B, H), jnp.int32),
        ],
        compiler_params=pltpu.CompilerParams(
            dimension_semantics=("arbitrary",)),
        interpret=interpret,
    )(x, pre_bias.reshape(1, D), W_enc, b_enc.reshape(1, H),
      post_bias.reshape(1, H))
    return enc, pen, bc


def _decode_dense(enc, W_dec, b_dec, dec_bias, interpret=False):
    return pl.pallas_call(
        _dec_kernel,
        grid=(NB,),
        in_specs=[
            pl.BlockSpec((B, BH), lambda i: (0, i)),
            pl.BlockSpec((BH, D), lambda i: (i, 0)),
            pl.BlockSpec((1, D), lambda i: (0, 0)),
            pl.BlockSpec((1, D), lambda i: (0, 0)),
        ],
        out_specs=pl.BlockSpec((B, D), lambda i: (0, 0)),
        out_shape=jax.ShapeDtypeStruct((B, D), jnp.float32),
        scratch_shapes=[pltpu.VMEM((B, D), jnp.float32)],
        compiler_params=pltpu.CompilerParams(
            dimension_semantics=("arbitrary",)),
        interpret=interpret,
    )(enc, W_dec, b_dec.reshape(1, D), dec_bias.reshape(1, D))


_L = 16
_CAP = 48


def _sc_dec_body(enc_hbm, wd_hbm, bd_hbm, db_hbm, bc_hbm, out_hbm,
                 row_v, idx_v, val_v, blk_v, bc_v, rows_v, rows_b,
                 bd_v, db_v, sem, sem_b):
    row = lax.axis_index("s") * 2 + lax.axis_index("c")
    cp_row = pltpu.make_async_copy(enc_hbm.at[row], row_v, sem_b)
    cp_row.start()
    pltpu.sync_copy(bc_hbm.at[row], bc_v)
    pltpu.sync_copy(bd_hbm, bd_v)
    pltpu.sync_copy(db_hbm, db_v)
    zi = jnp.zeros((_L,), jnp.int32)
    for j in range(_CAP // _L):
        idx_v[pl.ds(j * _L, _L)] = zi
        val_v[pl.ds(j * _L, _L)] = jnp.zeros((_L,), jnp.float32)

    def l1_body(i, nb):
        c = bc_v[pl.ds(i * _L, _L)]
        m = c > 0.5
        plsc.store_compressed(blk_v.at[pl.ds(nb, _L)],
                              lax.iota(jnp.int32, _L) + i * _L, mask=m)
        pc = plsc.all_reduce_population_count(m)[0]
        return jnp.minimum(nb + pc, _CAP - _L)

    nblk = lax.fori_loop(0, _NBLK // _L, l1_body, jnp.int32(0))
    cp_row.wait()

    def l2_body(j, cnt):
        ch = blk_v[pl.ds((j // _L) * _L, _L)]
        sel = jnp.full((_L,), j % _L, jnp.int32)
        bid = ch.at[sel].get(mode="promise_in_bounds")[0]
        base = bid * _BLK

        def inner(q, cnt):
            v = row_v[pl.ds(base + q * _L, _L)]
            m = v > 0.0
            plsc.store_compressed(val_v.at[pl.ds(cnt, _L)], v, mask=m)
            plsc.store_compressed(idx_v.at[pl.ds(cnt, _L)],
                                  lax.iota(jnp.int32, _L) + base + q * _L,
                                  mask=m)
            pc = plsc.all_reduce_population_count(m)[0]
            return jnp.minimum(cnt + pc, _CAP - _L)

        return lax.fori_loop(0, _BLK // _L, inner, cnt, unroll=True)

    lax.fori_loop(0, nblk, l2_body, jnp.int32(0))

    cp_a = pltpu.make_async_copy(wd_hbm.at[idx_v.at[pl.ds(0, K // 2)]],
                                 rows_v, sem)
    cp_b = pltpu.make_async_copy(wd_hbm.at[idx_v.at[pl.ds(K // 2, K // 2)]],
                                 rows_b, sem_b)
    cp_a.start()
    cp_b.start()

    vals = [val_v[pl.ds(j * _L, _L)][i]
            for j in range(K // _L) for i in range(_L)]

    cp_a.wait()

    def acc_a(c, carry):
        a = bd_v[pl.ds(c * _L, _L)] + db_v[pl.ds(c * _L, _L)]
        for k in range(K // 2):
            a = a + vals[k] * rows_v[k, pl.ds(c * _L, _L)]
        row_v[pl.ds(c * _L, _L)] = a
        return carry

    lax.fori_loop(0, D // _L, acc_a, 0)
    cp_b.wait()

    def acc_b(c, carry):
        a = row_v[pl.ds(c * _L, _L)]
        for k in range(K // 2):
            a = a + vals[K // 2 + k] * rows_b[k, pl.ds(c * _L, _L)]
        row_v[pl.ds(c * _L, _L)] = a
        return carry

    lax.fori_loop(0, D // _L, acc_b, 0)
    pltpu.sync_copy(row_v.at[pl.ds(0, D)], out_hbm.at[row])


def _decode_sparse(enc, W_dec, b_dec, dec_bias, bc):
    mesh = plsc.VectorSubcoreMesh(core_axis_name="c", subcore_axis_name="s")
    f = functools.partial(
        pl.kernel,
        out_type=jax.ShapeDtypeStruct((B, D), jnp.float32),
        mesh=mesh,
        scratch_types=[
            pltpu.VMEM((H,), jnp.float32),
            pltpu.VMEM((_CAP,), jnp.int32),
            pltpu.VMEM((_CAP,), jnp.float32),
            pltpu.VMEM((_CAP,), jnp.int32),
            pltpu.VMEM((_NBLK,), jnp.float32),
            pltpu.VMEM((K // 2, D), jnp.float32),
            pltpu.VMEM((K // 2, D), jnp.float32),
            pltpu.VMEM((D,), jnp.float32),
            pltpu.VMEM((D,), jnp.float32),
            pltpu.SemaphoreType.DMA,
            pltpu.SemaphoreType.DMA,
        ],
        compiler_params=pltpu.CompilerParams(needs_layout_passes=False),
    )(_sc_dec_body)
    return f(enc, W_dec, b_dec, dec_bias, bc)


def kernel(x, W_enc, b_enc, pre_bias, post_bias, W_dec, b_dec, dec_bias):
    enc, pen, bc = _encode(x, W_enc, b_enc, pre_bias, post_bias)
    decoded = _decode_sparse(enc, W_dec, b_dec, dec_bias, bc)
    return decoded, enc, pen.reshape(())
